# Initial kernel scaffold; baseline (speedup 1.0000x reference)
import jax
import jax.numpy as jnp
from jax.experimental import pallas as pl


def kernel(x, stem_w, stem_b, stem_g, stem_be, g1_fc1_w, g1_fc1_b, g1_fc1_g, g1_fc1_be, g1_gc_w, g1_gc_b, g1_gc_g, g1_gc_be, g1_fc2_w, g1_fc2_b, g1_fc2_g, g1_fc2_be, g2_fc1_w, g2_fc1_b, g2_fc1_g, g2_fc1_be, g2_gc_w, g2_gc_b, g2_gc_g, g2_gc_be, g2_fc2_w, g2_fc2_b, g2_fc2_g, g2_fc2_be, ds_w, ds_b, ds_g, ds_be):
    raise NotImplementedError("write your pallas kernel here")


# speedup vs baseline: 11.5362x; 11.5362x over previous
import functools

import jax
import jax.numpy as jnp
from jax import lax
from jax.experimental import pallas as pl
from jax.experimental.pallas import tpu as pltpu
from jax.experimental.pallas import tpu_sc as plsc

_K = 9
_EPS = 1e-5

_SC_NC = 2
_SC_NS = 16
_SC_NW = _SC_NC * _SC_NS


def _dot_bf16(a, b):
    return lax.dot_general(a.astype(jnp.bfloat16), b.astype(jnp.bfloat16),
                           (((1,), (0,)), ((), ())),
                           preferred_element_type=jnp.float32)


def _r2(v):
    return v.reshape(1, -1)


def _conv_body(xp_ref, w_ref, b_ref, o_ref):
    o_ref[...] = _dot_bf16(xp_ref[...], w_ref[...]) + b_ref[...]


def _conv_mm(xp, wT, b):
    n = xp.shape[0]
    c = wT.shape[1]
    return pl.pallas_call(
        _conv_body, out_shape=jax.ShapeDtypeStruct((n, c), jnp.float32),
    )(xp, wT, _r2(b))


def _bnmm_body(x_ref, m_ref, s_ref, g_ref, be_ref, w_ref, b_ref,
               y_ref, o_ref):
    y = (x_ref[...] - m_ref[...]) / s_ref[...] * g_ref[...] + be_ref[...]
    y = jnp.maximum(y, 0.0)
    y_ref[...] = y
    o_ref[...] = _dot_bf16(y, w_ref[...]) + b_ref[...]


def _bnrelu_mm(x, m, s, g, be, wT, b):
    n = x.shape[0]
    c = wT.shape[1]
    return pl.pallas_call(
        _bnmm_body,
        out_shape=(jax.ShapeDtypeStruct(x.shape, jnp.float32),
                   jax.ShapeDtypeStruct((n, c), jnp.float32)),
    )(x, m, s, _r2(g), _r2(be), wT, _r2(b))


def _bn_body(x_ref, m_ref, s_ref, g_ref, be_ref, o_ref):
    o_ref[...] = (x_ref[...] - m_ref[...]) / s_ref[...] * g_ref[...] \
        + be_ref[...]


def _bn_norm(x, m, s, g, be):
    return pl.pallas_call(
        _bn_body, out_shape=jax.ShapeDtypeStruct(x.shape, jnp.float32),
    )(x, m, s, _r2(g), _r2(be))


def _knn_body(xt_ref, xtT_ref, sqc_ref, sqr_ref, idx_ref):
    xt_t = xt_ref[...]
    mm = _dot_bf16(xt_t, xtT_ref[...])
    d = (sqc_ref[...] + sqr_ref[...]) - 2.0 * mm
    r, n = d.shape
    col = lax.broadcasted_iota(jnp.int32, (r, n), 1)
    big = jnp.int32(2 ** 30)
    inf = jnp.float32(jnp.inf)
    cols = []
    for _ in range(_K):
        m = jnp.min(d, axis=1, keepdims=True)
        sel = jnp.min(jnp.where(d == m, col, big), axis=1, keepdims=True)
        cols.append(sel)
        d = jnp.where(col == sel, inf, d)
    idx_ref[...] = jnp.concatenate(cols, axis=1)


def _knn(xt, xtT, sq, rows):
    n, c = xt.shape
    return pl.pallas_call(
        _knn_body,
        grid=(n // rows,),
        in_specs=[pl.BlockSpec((rows, c), lambda i: (i, 0)),
                  pl.BlockSpec((c, n), lambda i: (0, 0)),
                  pl.BlockSpec((rows, 1), lambda i: (i, 0)),
                  pl.BlockSpec((1, n), lambda i: (0, 0))],
        out_specs=pl.BlockSpec((rows, _K), lambda i: (i, 0)),
        out_shape=jax.ShapeDtypeStruct((n, _K), jnp.int32),
    )(xt, xtT, sq.reshape(n, 1), sq.reshape(1, n))


def _make_sc_gather(v, d, b, chunk):
    b_per_w = b // _SC_NW
    nch = b_per_w // chunk
    mesh = plsc.VectorSubcoreMesh(core_axis_name="c", subcore_axis_name="s")

    @functools.partial(
        pl.kernel, mesh=mesh,
        out_type=jax.ShapeDtypeStruct((b, d), jnp.float32),
        scratch_types=[pltpu.VMEM((nch, chunk), jnp.int32),
                       pltpu.VMEM((b_per_w, d), jnp.float32),
                       pltpu.SemaphoreType.DMA],
        compiler_params=pltpu.CompilerParams(use_tc_tiling_on_sc=False),
    )
    def gather(table_hbm, idx_hbm, out_hbm, idx_v, rows_v, sem):
        wid = lax.axis_index("s") * _SC_NC + lax.axis_index("c")
        pltpu.sync_copy(idx_hbm.at[wid], idx_v)
        copies = []
        for j in range(nch):
            copies.append(pltpu.async_copy(
                table_hbm.at[idx_v.at[j]],
                rows_v.at[pl.ds(j * chunk, chunk)], sem))
        for cp in copies:
            cp.wait()
        pltpu.sync_copy(rows_v, out_hbm.at[pl.ds(wid * b_per_w, b_per_w)])

    return gather


def _gc_body(nbrs_ref, xt_ref, w_ref, b_ref, o_ref):
    xt = xt_ref[...]
    mxn = nbrs_ref[0]
    for k in range(1, _K):
        mxn = jnp.maximum(mxn, nbrs_ref[k])
    h = jnp.concatenate([xt, mxn - xt], axis=-1)
    o_ref[...] = _dot_bf16(h, w_ref[...]) + b_ref[...]


def _gc_mm(nbrs, xt, gwT, gb):
    n = xt.shape[0]
    c2 = gwT.shape[1]
    return pl.pallas_call(
        _gc_body, out_shape=jax.ShapeDtypeStruct((n, c2), jnp.float32),
    )(nbrs, xt, gwT, _r2(gb))


def _out_body(x_ref, m_ref, s_ref, g_ref, be_ref, y_ref, o_ref):
    r = (x_ref[...] - m_ref[...]) / s_ref[...] * g_ref[...] + be_ref[...]
    o_ref[...] = jnp.maximum(r + y_ref[...], 0.0)


def _bn_add_relu(x, m, s, g, be, y):
    return pl.pallas_call(
        _out_body, out_shape=jax.ShapeDtypeStruct(x.shape, jnp.float32),
    )(x, m, s, _r2(g), _r2(be), y)


def _bn5_stats(y0, spatial):
    c = y0.shape[1]
    y5 = y0.T.reshape((1, c) + spatial)
    m = jnp.mean(y5, axis=(0, 2, 3, 4), keepdims=True)
    v = jnp.var(y5, axis=(0, 2, 3, 4), keepdims=True)
    return m.reshape(1, c), jnp.sqrt(v + _EPS).reshape(1, c)


def _tok_stats(x):
    m = jnp.mean(x[None], axis=(0, 1), keepdims=True)
    v = jnp.var(x[None], axis=(0, 1), keepdims=True)
    return m[0], jnp.sqrt(v + _EPS)[0]


def _stage(xp, w0T, b0, g0, be0, p, pref, spatial, chunk):
    n = xp.shape[0]
    y0 = _conv_mm(xp, w0T, b0)
    m0, s0 = _bn5_stats(y0, spatial)
    y, xt0 = _bnrelu_mm(y0, m0, s0, g0, be0,
                        p[pref + '_fc1_w'].T, p[pref + '_fc1_b'])
    m1, s1 = _tok_stats(xt0)
    xt = _bn_norm(xt0, m1, s1, p[pref + '_fc1_g'], p[pref + '_fc1_be'])
    sq = jnp.sum(xt[None] * xt[None], axis=-1)[0]
    idx = _knn(xt, xt.T, sq, 256)
    flat = idx.T.reshape(_SC_NW, -1, chunk)
    nbrs = _make_sc_gather(n, xt.shape[1], n * _K, chunk)(xt, flat)
    nbrs = nbrs.reshape(_K, n, xt.shape[1])
    h0 = _gc_mm(nbrs, xt, p[pref + '_gc_w'].T, p[pref + '_gc_b'])
    m2, s2 = _tok_stats(h0)
    _, r0 = _bnrelu_mm(h0, m2, s2, p[pref + '_gc_g'], p[pref + '_gc_be'],
                       p[pref + '_fc2_w'].T, p[pref + '_fc2_b'])
    m3, s3 = _tok_stats(r0)
    return _bn_add_relu(r0, m3, s3, p[pref + '_fc2_g'], p[pref + '_fc2_be'], y)


def kernel(x, stem_w, stem_b, stem_g, stem_be,
           g1_fc1_w, g1_fc1_b, g1_fc1_g, g1_fc1_be,
           g1_gc_w, g1_gc_b, g1_gc_g, g1_gc_be,
           g1_fc2_w, g1_fc2_b, g1_fc2_g, g1_fc2_be,
           g2_fc1_w, g2_fc1_b, g2_fc1_g, g2_fc1_be,
           g2_gc_w, g2_gc_b, g2_gc_g, g2_gc_be,
           g2_fc2_w, g2_fc2_b, g2_fc2_g, g2_fc2_be,
           ds_w, ds_b, ds_g, ds_be):
    p = dict(
        g1_fc1_w=g1_fc1_w, g1_fc1_b=g1_fc1_b, g1_fc1_g=g1_fc1_g,
        g1_fc1_be=g1_fc1_be, g1_gc_w=g1_gc_w, g1_gc_b=g1_gc_b,
        g1_gc_g=g1_gc_g, g1_gc_be=g1_gc_be, g1_fc2_w=g1_fc2_w,
        g1_fc2_b=g1_fc2_b, g1_fc2_g=g1_fc2_g, g1_fc2_be=g1_fc2_be,
        g2_fc1_w=g2_fc1_w, g2_fc1_b=g2_fc1_b, g2_fc1_g=g2_fc1_g,
        g2_fc1_be=g2_fc1_be, g2_gc_w=g2_gc_w, g2_gc_b=g2_gc_b,
        g2_gc_g=g2_gc_g, g2_gc_be=g2_gc_be, g2_fc2_w=g2_fc2_w,
        g2_fc2_b=g2_fc2_b, g2_fc2_g=g2_fc2_g, g2_fc2_be=g2_fc2_be,
    )

    xs = x.reshape(16, 4, 16, 4, 16, 4).transpose(0, 2, 4, 1, 3, 5)
    xp1 = xs.reshape(4096, 64)
    w0T = stem_w.reshape(48, 64).T

    f1 = _stage(xp1, w0T, stem_b, stem_g, stem_be, p, 'g1',
                (16, 16, 16), 128)

    fs = f1.reshape(8, 2, 8, 2, 8, 2, 48).transpose(0, 2, 4, 1, 3, 5, 6)
    xp2 = fs.reshape(512, 384)
    dsT = ds_w.transpose(2, 3, 4, 1, 0).reshape(384, 96)

    f2 = _stage(xp2, dsT, ds_b, ds_g, ds_be, p, 'g2', (8, 8, 8), 72)

    f1_out = f1.T.reshape(1, 48, 16, 16, 16)
    f2_out = f2.T.reshape(1, 96, 8, 8, 8)
    return (f1_out, f2_out)
